# baseline (speedup 1.0000x reference)
import functools

import jax
import jax.numpy as jnp
from jax import lax
from jax.experimental import pallas as pl
from jax.experimental.pallas import tpu as pltpu
from jax.experimental.pallas import tpu_sc as plsc

_NUM_CORES = 2
_NUM_SUBCORES = 16
_NUM_WORKERS = _NUM_CORES * _NUM_SUBCORES
_C = 80
_L = 16
_D = 100
_PL = 7
_DO = 300


def _embed_lookup(words, tags, lemmas, word_table, tag_table, lemma_table):
    n = words.shape[0]
    per_w = n // _NUM_WORKERS
    n_chunks = per_w // _C
    n_pairs = n_chunks // 2
    assert per_w * _NUM_WORKERS == n and n_chunks * _C == per_w
    assert n_pairs * 2 == n_chunks and n_pairs >= 2

    mesh = plsc.VectorSubcoreMesh(core_axis_name="c", subcore_axis_name="s")

    @functools.partial(
        pl.kernel,
        out_type=jax.ShapeDtypeStruct((1, n * _DO), jnp.float32),
        mesh=mesh,
        compiler_params=pltpu.CompilerParams(
            use_tc_tiling_on_sc=False, needs_layout_passes=False),
        scratch_types=[
            pltpu.VMEM((3, per_w), jnp.int32),
            pltpu.VMEM((2, 3, _PL, _C, _L), jnp.float32),
            pltpu.VMEM((2, 3 * _PL, 1, _C), jnp.int32),
            pltpu.VMEM((1, _C * _DO), jnp.float32),
            pltpu.SemaphoreType.DMA,
            pltpu.SemaphoreType.DMA,
        ],
    )
    def k(words_h, tags_h, lemmas_h, wt_h, tt_h, lt_h, out_h,
          iv, gbuf, pidx, comb, semA, semB):
        wid = lax.axis_index("s") * _NUM_CORES + lax.axis_index("c")
        base = wid * per_w
        lane = lax.iota(jnp.int32, _L)
        sems = (semA, semB)
        tbls = (wt_h, tt_h, lt_h)
        srcs = (words_h, tags_h, lemmas_h)

        for b in range(3):
            pltpu.sync_copy(srcs[b].at[pl.ds(base, per_w)], iv.at[b])

        def fire(c, sl):
            off = c * _C
            for b in range(3):
                for g in range(_C // _L):
                    v = iv[b, pl.ds(off + g * _L, _L)]
                    j0 = _PL * v
                    for r in range(_PL):
                        pidx[sl, b * _PL + r, 0, pl.ds(g * _L, _L)] = j0 + r
            for b in range(3):
                for r in range(_PL):
                    pltpu.async_copy(
                        tbls[b].at[pidx.at[sl, b * _PL + r, 0]],
                        gbuf.at[sl, b, r], sems[sl])

        def drain(sl):
            for b in range(3):
                for r in range(_PL):
                    pltpu.make_async_copy(
                        tbls[b].at[pidx.at[sl, b * _PL + r, 0]],
                        gbuf.at[sl, b, r], sems[sl]).wait()

        def asm_store(c, sl):
            zero = lane * 0
            mtail = lane < (_D - (_PL - 1) * _L)

            @pl.loop(0, _C // _L)
            def asm(g):
                for b in range(3):
                    for j in range(_L):
                        kk = g * _L + j
                        dst0 = kk * _DO + b * _D
                        for r in range(_PL - 1):
                            comb[0, pl.ds(dst0 + r * _L, _L)] = \
                                gbuf[sl, b, r, kk, :]
                        x = gbuf[sl, b, _PL - 1, kk, :]
                        idx = (dst0 + (_PL - 1) * _L) + lane
                        plsc.store_scatter(comb, [zero, idx], x, mask=mtail)

            pltpu.sync_copy(
                comb, out_h.at[:, pl.ds((base + c * _C) * _DO, _C * _DO)])

        fire(0, 0)

        @pl.loop(0, n_pairs - 1)
        def pair(h):
            c0 = 2 * h
            fire(c0 + 1, 1)
            drain(0)
            asm_store(c0, 0)
            fire(c0 + 2, 0)
            drain(1)
            asm_store(c0 + 1, 1)

        c0 = n_chunks - 2
        fire(c0 + 1, 1)
        drain(0)
        asm_store(c0, 0)
        drain(1)
        asm_store(c0 + 1, 1)

    return k(words, tags, lemmas, word_table, tag_table, lemma_table)


def _prep(t):
    pad = _PL * _L - t.shape[1]
    return jnp.pad(t, ((0, 0), (0, pad))).reshape(-1, _L)


def kernel(words, tags, lemmas, word_table, tag_table, lemma_table):
    b, l = words.shape
    d = word_table.shape[1] + tag_table.shape[1] + lemma_table.shape[1]
    out = _embed_lookup(
        words.reshape(-1), tags.reshape(-1), lemmas.reshape(-1),
        _prep(word_table), _prep(tag_table), _prep(lemma_table))
    return out.reshape(b, l, d)


# speedup vs baseline: 1.0526x; 1.0526x over previous
import functools

import jax
import jax.numpy as jnp
from jax import lax
from jax.experimental import pallas as pl
from jax.experimental.pallas import tpu as pltpu
from jax.experimental.pallas import tpu_sc as plsc

_NUM_CORES = 2
_NUM_SUBCORES = 16
_NUM_WORKERS = _NUM_CORES * _NUM_SUBCORES
_L = 16
_D = 100
_DP = 128
_DO = 300
_C1 = 64
_C2 = 80


def _gather_padded(words, tags, lemmas, wt, tt, lt):
    n = words.shape[0]
    per_w = n // _NUM_WORKERS
    n_chunks = per_w // _C1
    assert n_chunks * _C1 * _NUM_WORKERS == n and n_chunks % 2 == 0

    mesh = plsc.VectorSubcoreMesh(core_axis_name="c", subcore_axis_name="s")

    @functools.partial(
        pl.kernel,
        out_type=jax.ShapeDtypeStruct((3, n, _DP), jnp.float32),
        mesh=mesh,
        scratch_types=[
            pltpu.VMEM((per_w,), jnp.int32),
            pltpu.VMEM((per_w,), jnp.int32),
            pltpu.VMEM((per_w,), jnp.int32),
            pltpu.VMEM((2, 3, _C1, _DP), jnp.float32),
            pltpu.SemaphoreType.DMA,
            pltpu.SemaphoreType.DMA,
        ],
    )
    def k1(words_h, tags_h, lemmas_h, wt_h, tt_h, lt_h, out_h,
           iw, it, il, gbuf, semA, semB):
        wid = lax.axis_index("s") * _NUM_CORES + lax.axis_index("c")
        base = wid * per_w
        sems = (semA, semB)
        tbls = (wt_h, tt_h, lt_h)
        srcs = (words_h, tags_h, lemmas_h)
        ivs = (iw, it, il)
        for b in range(3):
            pltpu.sync_copy(srcs[b].at[pl.ds(base, per_w)], ivs[b])

        def fire(c, sl):
            for b in range(3):
                pltpu.async_copy(
                    tbls[b].at[ivs[b].at[pl.ds(c * _C1, _C1)]],
                    gbuf.at[sl, b], sems[sl])

        def drain_store(c, sl):
            for b in range(3):
                pltpu.make_async_copy(
                    tbls[b].at[ivs[b].at[pl.ds(c * _C1, _C1)]],
                    gbuf.at[sl, b], sems[sl]).wait()
            for b in range(3):
                pltpu.sync_copy(
                    gbuf.at[sl, b],
                    out_h.at[b, pl.ds(base + c * _C1, _C1), :])

        fire(0, 0)

        @pl.loop(0, n_chunks // 2 - 1)
        def pair(h):
            c0 = 2 * h
            fire(c0 + 1, 1)
            drain_store(c0, 0)
            fire(c0 + 2, 0)
            drain_store(c0 + 1, 1)

        c0 = n_chunks - 2
        fire(c0 + 1, 1)
        drain_store(c0, 0)
        drain_store(c0 + 1, 1)

    return k1(words, tags, lemmas, wt, tt, lt)


def _interleave(g, n):
    per_w = n // _NUM_WORKERS
    n_chunks = per_w // _C2
    assert n_chunks * _C2 * _NUM_WORKERS == n and n_chunks % 2 == 0

    mesh = plsc.VectorSubcoreMesh(core_axis_name="c", subcore_axis_name="s")

    @functools.partial(
        pl.kernel,
        out_type=jax.ShapeDtypeStruct((1, n * _DO), jnp.float32),
        mesh=mesh,
        compiler_params=pltpu.CompilerParams(
            use_tc_tiling_on_sc=False, needs_layout_passes=False),
        scratch_types=[
            pltpu.VMEM((2, 3, _C2, _DP), jnp.float32),
            pltpu.VMEM((1, _C2 * _DO), jnp.float32),
            pltpu.SemaphoreType.DMA,
            pltpu.SemaphoreType.DMA,
        ],
    )
    def k2(g_h, out_h, gbuf, comb, semA, semB):
        wid = lax.axis_index("s") * _NUM_CORES + lax.axis_index("c")
        base = wid * per_w
        lane = lax.iota(jnp.int32, _L)
        sems = (semA, semB)
        zero = lane * 0
        mtail = lane < (_D - 6 * _L)

        def fire(c, sl):
            for b in range(3):
                pltpu.async_copy(
                    g_h.at[b, pl.ds(base + c * _C2, _C2), :],
                    gbuf.at[sl, b], sems[sl])

        def drain(c, sl):
            for b in range(3):
                pltpu.make_async_copy(
                    g_h.at[b, pl.ds(base + c * _C2, _C2), :],
                    gbuf.at[sl, b], sems[sl]).wait()

        def asm_store(c, sl):
            @pl.loop(0, _C2 // _L)
            def asm(g_):
                for b in range(3):
                    for j in range(_L):
                        kk = g_ * _L + j
                        dst0 = kk * _DO + b * _D
                        for r in range(6):
                            comb[0, pl.ds(dst0 + r * _L, _L)] = \
                                gbuf[sl, b, kk, pl.ds(r * _L, _L)]
                        x = gbuf[sl, b, kk, pl.ds(6 * _L, _L)]
                        idx = (dst0 + 6 * _L) + lane
                        plsc.store_scatter(comb, [zero, idx], x, mask=mtail)

            pltpu.sync_copy(
                comb, out_h.at[:, pl.ds((base + c * _C2) * _DO, _C2 * _DO)])

        fire(0, 0)

        @pl.loop(0, n_chunks // 2 - 1)
        def pair(h):
            c0 = 2 * h
            fire(c0 + 1, 1)
            drain(c0, 0)
            asm_store(c0, 0)
            fire(c0 + 2, 0)
            drain(c0 + 1, 1)
            asm_store(c0 + 1, 1)

        c0 = n_chunks - 2
        fire(c0 + 1, 1)
        drain(c0, 0)
        asm_store(c0, 0)
        drain(c0 + 1, 1)
        asm_store(c0 + 1, 1)

    return k2(g)


def kernel(words, tags, lemmas, word_table, tag_table, lemma_table):
    b, l = words.shape
    d = word_table.shape[1] + tag_table.shape[1] + lemma_table.shape[1]
    n = b * l

    def pad128(t):
        return jnp.pad(t, ((0, 0), (0, _DP - t.shape[1])))

    g = _gather_padded(
        words.reshape(-1), tags.reshape(-1), lemmas.reshape(-1),
        pad128(word_table), pad128(tag_table), pad128(lemma_table))
    out = _interleave(g, n)
    return out.reshape(b, l, d)


# speedup vs baseline: 2.1834x; 2.0743x over previous
import functools

import jax
import jax.numpy as jnp
from jax import lax
from jax.experimental import pallas as pl
from jax.experimental.pallas import tpu as pltpu
from jax.experimental.pallas import tpu_sc as plsc

_NUM_CORES = 2
_NUM_SUBCORES = 16
_NUM_WORKERS = _NUM_CORES * _NUM_SUBCORES
_L = 16
_D = 100
_DP = 128
_DO = 300
_C1 = 64
_C2 = 80


def _gather_padded(words, tags, lemmas, wt, tt, lt):
    n = words.shape[0]
    per_w = n // _NUM_WORKERS
    n_chunks = per_w // _C1
    assert n_chunks * _C1 * _NUM_WORKERS == n and n_chunks % 2 == 0

    mesh = plsc.VectorSubcoreMesh(core_axis_name="c", subcore_axis_name="s")

    @functools.partial(
        pl.kernel,
        out_type=jax.ShapeDtypeStruct((3, n, _DP), jnp.float32),
        mesh=mesh,
        scratch_types=[
            pltpu.VMEM((per_w,), jnp.int32),
            pltpu.VMEM((per_w,), jnp.int32),
            pltpu.VMEM((per_w,), jnp.int32),
            pltpu.VMEM((2, 3, _C1, _DP), jnp.float32),
            pltpu.SemaphoreType.DMA,
            pltpu.SemaphoreType.DMA,
        ],
    )
    def k1(words_h, tags_h, lemmas_h, wt_h, tt_h, lt_h, out_h,
           iw, it, il, gbuf, semA, semB):
        wid = lax.axis_index("s") * _NUM_CORES + lax.axis_index("c")
        base = wid * per_w
        sems = (semA, semB)
        tbls = (wt_h, tt_h, lt_h)
        srcs = (words_h, tags_h, lemmas_h)
        ivs = (iw, it, il)
        for b in range(3):
            pltpu.sync_copy(srcs[b].at[pl.ds(base, per_w)], ivs[b])

        def fire(c, sl):
            for b in range(3):
                pltpu.async_copy(
                    tbls[b].at[ivs[b].at[pl.ds(c * _C1, _C1)]],
                    gbuf.at[sl, b], sems[sl])

        def drain_store(c, sl):
            for b in range(3):
                pltpu.make_async_copy(
                    tbls[b].at[ivs[b].at[pl.ds(c * _C1, _C1)]],
                    gbuf.at[sl, b], sems[sl]).wait()
            for b in range(3):
                pltpu.sync_copy(
                    gbuf.at[sl, b],
                    out_h.at[b, pl.ds(base + c * _C1, _C1), :])

        fire(0, 0)

        @pl.loop(0, n_chunks // 2 - 1)
        def pair(h):
            c0 = 2 * h
            fire(c0 + 1, 1)
            drain_store(c0, 0)
            fire(c0 + 2, 0)
            drain_store(c0 + 1, 1)

        c0 = n_chunks - 2
        fire(c0 + 1, 1)
        drain_store(c0, 0)
        drain_store(c0 + 1, 1)

    return k1(words, tags, lemmas, wt, tt, lt)


def _interleave(g, n):
    per_w = n // _NUM_WORKERS
    n_chunks = per_w // _C2
    assert n_chunks * _C2 * _NUM_WORKERS == n and n_chunks % 2 == 0

    mesh = plsc.VectorSubcoreMesh(core_axis_name="c", subcore_axis_name="s")

    @functools.partial(
        pl.kernel,
        out_type=jax.ShapeDtypeStruct((n * _DO,), jnp.float32),
        mesh=mesh,
        compiler_params=pltpu.CompilerParams(
            use_tc_tiling_on_sc=False, needs_layout_passes=False),
        scratch_types=[
            pltpu.VMEM((2, 3, _C2, _DP), jnp.float32),
            pltpu.VMEM((1, _C2 * _DO), jnp.float32),
            pltpu.SemaphoreType.DMA,
            pltpu.SemaphoreType.DMA,
        ],
    )
    def k2(g_h, out_h, gbuf, comb, semA, semB):
        wid = lax.axis_index("s") * _NUM_CORES + lax.axis_index("c")
        base = wid * per_w
        lane = lax.iota(jnp.int32, _L)
        sems = (semA, semB)
        zero = lane * 0
        mtail = lane < (_D - 6 * _L)

        def fire(c, sl):
            for b in range(3):
                pltpu.async_copy(
                    g_h.at[b, pl.ds(base + c * _C2, _C2), :],
                    gbuf.at[sl, b], sems[sl])

        def drain(c, sl):
            for b in range(3):
                pltpu.make_async_copy(
                    g_h.at[b, pl.ds(base + c * _C2, _C2), :],
                    gbuf.at[sl, b], sems[sl]).wait()

        def asm_store(c, sl):
            @pl.loop(0, _C2 // _L)
            def asm(g_):
                for b in range(3):
                    for j in range(_L):
                        kk = g_ * _L + j
                        dst0 = kk * _DO + b * _D
                        for r in range(6):
                            comb[0, pl.ds(dst0 + r * _L, _L)] = \
                                gbuf[sl, b, kk, pl.ds(r * _L, _L)]
                        x = gbuf[sl, b, kk, pl.ds(6 * _L, _L)]
                        idx = (dst0 + 6 * _L) + lane
                        plsc.store_scatter(comb, [zero, idx], x, mask=mtail)

            pltpu.sync_copy(
                comb.at[0],
                out_h.at[pl.ds((base + c * _C2) * _DO, _C2 * _DO)])

        fire(0, 0)

        @pl.loop(0, n_chunks // 2 - 1)
        def pair(h):
            c0 = 2 * h
            fire(c0 + 1, 1)
            drain(c0, 0)
            asm_store(c0, 0)
            fire(c0 + 2, 0)
            drain(c0 + 1, 1)
            asm_store(c0 + 1, 1)

        c0 = n_chunks - 2
        fire(c0 + 1, 1)
        drain(c0, 0)
        asm_store(c0, 0)
        drain(c0 + 1, 1)
        asm_store(c0 + 1, 1)

    return k2(g)


def kernel(words, tags, lemmas, word_table, tag_table, lemma_table):
    b, l = words.shape
    d = word_table.shape[1] + tag_table.shape[1] + lemma_table.shape[1]
    n = b * l

    def pad128(t):
        return jnp.pad(t, ((0, 0), (0, _DP - t.shape[1])))

    g = _gather_padded(
        words.reshape(-1), tags.reshape(-1), lemmas.reshape(-1),
        pad128(word_table), pad128(tag_table), pad128(lemma_table))
    out = _interleave(g, n)
    return out.reshape(b, l, d)
